# initial kernel scaffold (unmeasured)
import jax
import jax.numpy as jnp
from jax import lax
from jax.experimental import pallas as pl
from jax.experimental.pallas import tpu as pltpu

N_Z = 4


def kernel(Q, K, V):
    b, s, h, d = Q.shape
    scale = d ** -0.5

    def body(q_ref, k_ref, v_ref, out_ref, kv_ref, send_sems, recv_sems):
        my_x = lax.axis_index("x")
        my_y = lax.axis_index("y")
        my_z = lax.axis_index("z")
        left = (my_z - 1) % N_Z
        right = (my_z + 1) % N_Z

        kv_ref[0, 0] = k_ref[...]
        kv_ref[0, 1] = v_ref[...]

        barrier_sem = pltpu.get_barrier_semaphore()
        for nbr in (left, right):
            pl.semaphore_signal(
                barrier_sem,
                inc=1,
                device_id=(my_x, my_y, nbr),
                device_id_type=pl.DeviceIdType.MESH,
            )
        pl.semaphore_wait(barrier_sem, 2)

        for t in range(N_Z - 1):
            rdma = pltpu.make_async_remote_copy(
                src_ref=kv_ref.at[t],
                dst_ref=kv_ref.at[t + 1],
                send_sem=send_sems.at[t],
                recv_sem=recv_sems.at[t],
                device_id=(my_x, my_y, right),
                device_id_type=pl.DeviceIdType.MESH,
            )
            rdma.start()
            rdma.wait()

        for bi in range(b):
            for hi in range(h):
                q = q_ref[bi, :, hi, :]
                s_parts = []
                for j in range(N_Z):
                    kj = kv_ref[j, 0, bi, :, hi, :]
                    s_parts.append(
                        lax.dot_general(
                            q,
                            kj,
                            (((1,), (1,)), ((), ())),
                            preferred_element_type=jnp.float32,
                        )
                    )
                s_full = jnp.concatenate(s_parts, axis=1) * scale
                m = jnp.max(s_full, axis=1, keepdims=True)
                p = jnp.exp(s_full - m)
                p = p / jnp.sum(p, axis=1, keepdims=True)
                acc = jnp.zeros((s, d), jnp.float32)
                for j in range(N_Z):
                    vj = kv_ref[j, 1, bi, :, hi, :]
                    acc = acc + lax.dot_general(
                        p[:, j * s : (j + 1) * s],
                        vj,
                        (((1,), (0,)), ((), ())),
                        preferred_element_type=jnp.float32,
                    )
                out_ref[bi, :, hi, :] = acc

    return pl.pallas_call(
        body,
        out_shape=jax.ShapeDtypeStruct((b, s, h, d), jnp.float32),
        in_specs=[pl.BlockSpec(memory_space=pltpu.VMEM)] * 3,
        out_specs=pl.BlockSpec(memory_space=pltpu.VMEM),
        scratch_shapes=[
            pltpu.VMEM((N_Z, 2, b, s, h, d), jnp.float32),
            pltpu.SemaphoreType.DMA((N_Z - 1,)),
            pltpu.SemaphoreType.DMA((N_Z - 1,)),
        ],
        compiler_params=pltpu.CompilerParams(collective_id=0),
    )(Q, K, V)


# baseline (device time: 342285 ns/iter reference)
import jax
import jax.numpy as jnp
from jax import lax
from jax.experimental import pallas as pl
from jax.experimental.pallas import tpu as pltpu

N_Z = 4


def kernel(Q, K, V):
    b, s, h, d = Q.shape
    scale = d ** -0.5

    def body(q_ref, k_ref, v_ref, out_ref, kv_ref, send_sems, recv_sems):
        my_x = lax.axis_index("x")
        my_y = lax.axis_index("y")
        my_z = lax.axis_index("z")
        left = (my_z - 1) % N_Z
        right = (my_z + 1) % N_Z

        kv_ref[0, 0] = k_ref[...]
        kv_ref[0, 1] = v_ref[...]

        barrier_sem = pltpu.get_barrier_semaphore()
        for nbr in (left, right):
            pl.semaphore_signal(
                barrier_sem,
                inc=1,
                device_id=(my_x, my_y, nbr),
                device_id_type=pl.DeviceIdType.MESH,
            )
        pl.semaphore_wait(barrier_sem, 2)

        for t in range(N_Z - 1):
            rdma = pltpu.make_async_remote_copy(
                src_ref=kv_ref.at[t],
                dst_ref=kv_ref.at[t + 1],
                send_sem=send_sems.at[t],
                recv_sem=recv_sems.at[t],
                device_id=(my_x, my_y, right),
                device_id_type=pl.DeviceIdType.MESH,
            )
            rdma.start()
            rdma.wait()

        for bi in range(b):
            for hi in range(h):
                q = q_ref[bi, :, hi, :]
                s_parts = []
                for j in range(N_Z):
                    kj = kv_ref[j, 0, bi, :, hi, :]
                    s_parts.append(
                        lax.dot_general(
                            q,
                            kj,
                            (((1,), (1,)), ((), ())),
                            preferred_element_type=jnp.float32,
                        )
                    )
                s_full = jnp.concatenate(s_parts, axis=1) * scale
                m = jnp.max(s_full, axis=1, keepdims=True)
                p = jnp.exp(s_full - m)
                p = p / jnp.sum(p, axis=1, keepdims=True)
                acc = jnp.zeros((s, d), jnp.float32)
                for j in range(N_Z):
                    vj = kv_ref[j, 1, bi, :, hi, :]
                    acc = acc + lax.dot_general(
                        p[:, j * s : (j + 1) * s],
                        vj,
                        (((1,), (0,)), ((), ())),
                        preferred_element_type=jnp.float32,
                    )
                out_ref[bi, :, hi, :] = acc

    return pl.pallas_call(
        body,
        out_shape=jax.ShapeDtypeStruct((b, s, h, d), jnp.float32),
        in_specs=[pl.BlockSpec(memory_space=pltpu.VMEM)] * 3,
        out_specs=pl.BlockSpec(memory_space=pltpu.VMEM),
        scratch_shapes=[
            pltpu.VMEM((N_Z, 2, b, s, h, d), jnp.float32),
            pltpu.SemaphoreType.DMA((N_Z - 1,)),
            pltpu.SemaphoreType.DMA((N_Z - 1,)),
        ],
        compiler_params=pltpu.CompilerParams(
            collective_id=0,
            vmem_limit_bytes=100 * 1024 * 1024,
        ),
    )(Q, K, V)
